# baseline (speedup 1.0000x reference)
import jax
import jax.numpy as jnp
from jax.experimental import pallas as pl
from jax.experimental.pallas import tpu as pltpu

EPS_BN = 1e-5
SLOPE = 0.1

_OFFS = ((0, 1, 2), (1, 2))
_GROUPS = (((0,), (1, 2), (3,)), ((0, 1), (2, 3)))


def _leaky(v):
    return jnp.where(v >= 0, v, SLOPE * v)


def _stats_kernel(x_ref, s_ref, q_ref):
    xb = x_ref[0]
    s_ref[0, 0, :] = jnp.sum(xb, axis=1)
    q_ref[0, 0, :] = jnp.sum(xb * xb, axis=1)


def _conv1_kernel(x_ref, sc_ref, sh_ref, w00_ref, w01_ref, w10_ref, w11_ref,
                  b_ref, y_ref, xb_ref, s_ref, q_ref, *, h, w, co):
    xt = jnp.transpose(x_ref[0])
    xb_ref[0] = xt.astype(jnp.bfloat16)
    a = _leaky(xt * sc_ref[0].reshape(1, -1) + sh_ref[0].reshape(1, -1))
    a = a.astype(jnp.bfloat16).reshape(h, w, -1)
    a = jnp.concatenate([a[0:1], a, a[h - 1:h]], axis=0)
    a = jnp.concatenate([a[:, 0:1], a, a[:, w - 1:w]], axis=1)
    wrefs = ((w00_ref, w01_ref), (w10_ref, w11_ref))
    bias = b_ref[0].reshape(1, co)
    ssum = jnp.zeros((1, co), jnp.float32)
    ssq = jnp.zeros((1, co), jnp.float32)
    planes = [[None, None], [None, None]]
    for pa in (0, 1):
        for pb in (0, 1):
            pieces = [a[i:i + h, j:j + w, :]
                      for i in _OFFS[pa] for j in _OFFS[pb]]
            patch = jnp.concatenate(pieces, axis=-1).reshape(h * w, -1)
            acc = jnp.dot(patch, wrefs[pa][pb][...],
                          preferred_element_type=jnp.float32) + bias
            ssum = ssum + jnp.sum(acc, axis=0, keepdims=True)
            ssq = ssq + jnp.sum(acc * acc, axis=0, keepdims=True)
            planes[pa][pb] = acc.astype(jnp.bfloat16)
    s_ref[0] = ssum
    q_ref[0] = ssq
    c0 = jnp.concatenate([planes[0][0], planes[0][1]],
                         axis=1).reshape(h, 2 * w, co)
    c1 = jnp.concatenate([planes[1][0], planes[1][1]],
                         axis=1).reshape(h, 2 * w, co)
    z = jnp.concatenate([c0, c1], axis=1).reshape(2 * h, 2 * w, co)
    z = jnp.concatenate([z[:, 0:1], z, z[:, -1:], z[:, -1:]], axis=1)
    z = jnp.concatenate([z[0:1], z, z[-1:], z[-1:]], axis=0)
    y_ref[0] = z


def _conv2_kernel(y_ref, sc_ref, sh_ref, w2_ref, xb_ref, ws_ref, b_ref,
                  o_ref, *, th, w, wo, co):
    r = pl.program_id(1)
    row0 = pl.multiple_of(r * th, th)
    yt = y_ref[0, pl.ds(row0, th + 3)]
    a = _leaky(yt.astype(jnp.float32) * sc_ref[0].reshape(1, 1, -1)
               + sh_ref[0].reshape(1, 1, -1)).astype(jnp.bfloat16)
    pieces = [a[i:i + th, j:j + wo, :] for i in range(4) for j in range(4)]
    patch = jnp.concatenate(pieces, axis=-1).reshape(th * wo, -1)
    acc = jnp.dot(patch, w2_ref[...], preferred_element_type=jnp.float32)
    hs = th // 2
    xs = xb_ref[0, pl.ds(r * hs * w, hs * w)].reshape(hs, w, -1)
    xs = jnp.stack([xs, xs], axis=1).reshape(th, w, -1)
    xs = jnp.stack([xs, xs], axis=2).reshape(th, wo, -1).reshape(th * wo, -1)
    acc = acc + jnp.dot(xs, ws_ref[...], preferred_element_type=jnp.float32)
    acc = acc + b_ref[0].reshape(1, co)
    o_ref[0] = jnp.transpose(acc.reshape(th, wo, co), (2, 0, 1))


def _sn(wgt, u0, eps=1e-12):
    co = wgt.shape[0]
    wm = wgt.reshape(co, -1)
    u = u0 / jnp.maximum(jnp.linalg.norm(u0), eps)
    v = wm.T @ u
    v = v / jnp.maximum(jnp.linalg.norm(v), eps)
    u = wm @ v
    u = u / jnp.maximum(jnp.linalg.norm(u), eps)
    sigma = u @ (wm @ v)
    return wgt / sigma


def _parity_mat(wt, pa, pb):
    rows = jnp.stack([sum(wt[i] for i in g) for g in _GROUPS[pa]], axis=0)
    full = jnp.stack([sum(rows[:, j] for j in g) for g in _GROUPS[pb]],
                     axis=1)
    lr, lc, ci, co = full.shape
    return full.reshape(lr * lc * ci, co).astype(jnp.bfloat16)


def _bn_affine(ssum, ssq, count, gamma, beta):
    mean = ssum / count
    var = jnp.maximum(ssq / count - mean * mean, 0.0)
    inv = gamma * jax.lax.rsqrt(var + EPS_BN)
    return inv.reshape(1, -1), (beta - mean * inv).reshape(1, -1)


def kernel(x, bn1_gamma, bn1_beta, w1, b1, u1,
           bn2_gamma, bn2_beta, w2, b2, u2, ws, bs):
    n, ci, h, w = x.shape
    co = w1.shape[0]
    ho, wo = 2 * h, 2 * w
    xf = x.astype(jnp.float32).reshape(n, ci, h * w)

    s1p, q1p = pl.pallas_call(
        _stats_kernel,
        grid=(n,),
        in_specs=[pl.BlockSpec((1, ci, h * w), lambda i: (i, 0, 0))],
        out_specs=(pl.BlockSpec((1, 1, ci), lambda i: (i, 0, 0)),
                   pl.BlockSpec((1, 1, ci), lambda i: (i, 0, 0))),
        out_shape=(jax.ShapeDtypeStruct((n, 1, ci), jnp.float32),
                   jax.ShapeDtypeStruct((n, 1, ci), jnp.float32)),
        compiler_params=pltpu.CompilerParams(
            dimension_semantics=("parallel",)),
    )(xf)
    scale1, shift1 = _bn_affine(s1p.sum(axis=(0, 1)), q1p.sum(axis=(0, 1)),
                                n * h * w, bn1_gamma, bn1_beta)

    wt1 = jnp.transpose(_sn(w1, u1), (2, 3, 1, 0))
    wmats = [_parity_mat(wt1, pa, pb) for pa in (0, 1) for pb in (0, 1)]

    conv1 = pl.pallas_call(
        lambda *refs: _conv1_kernel(*refs, h=h, w=w, co=co),
        grid=(n,),
        in_specs=[
            pl.BlockSpec((1, ci, h * w), lambda b: (b, 0, 0)),
            pl.BlockSpec((1, ci), lambda b: (0, 0)),
            pl.BlockSpec((1, ci), lambda b: (0, 0)),
            pl.BlockSpec(wmats[0].shape, lambda b: (0, 0)),
            pl.BlockSpec(wmats[1].shape, lambda b: (0, 0)),
            pl.BlockSpec(wmats[2].shape, lambda b: (0, 0)),
            pl.BlockSpec(wmats[3].shape, lambda b: (0, 0)),
            pl.BlockSpec((1, co), lambda b: (0, 0)),
        ],
        out_specs=(
            pl.BlockSpec((1, ho + 3, wo + 3, co), lambda b: (b, 0, 0, 0)),
            pl.BlockSpec((1, h * w, ci), lambda b: (b, 0, 0)),
            pl.BlockSpec((1, 1, co), lambda b: (b, 0, 0)),
            pl.BlockSpec((1, 1, co), lambda b: (b, 0, 0)),
        ),
        out_shape=(
            jax.ShapeDtypeStruct((n, ho + 3, wo + 3, co), jnp.bfloat16),
            jax.ShapeDtypeStruct((n, h * w, ci), jnp.bfloat16),
            jax.ShapeDtypeStruct((n, 1, co), jnp.float32),
            jax.ShapeDtypeStruct((n, 1, co), jnp.float32),
        ),
        compiler_params=pltpu.CompilerParams(
            dimension_semantics=("parallel",)),
    )
    y1p, xb16, s2p, q2p = conv1(xf, scale1, shift1, *wmats, b1.reshape(1, co))

    scale2, shift2 = _bn_affine(s2p.sum(axis=(0, 1)), q2p.sum(axis=(0, 1)),
                                n * ho * wo, bn2_gamma, bn2_beta)

    w2m = jnp.transpose(_sn(w2, u2), (2, 3, 1, 0)).reshape(16 * co, co)
    w2m = w2m.astype(jnp.bfloat16)
    wsm = ws[:, :, 0, 0].T.astype(jnp.bfloat16)
    bias = (b2 + bs).reshape(1, co)

    th = 16 if ho % 16 == 0 else ho
    out = pl.pallas_call(
        lambda *refs: _conv2_kernel(*refs, th=th, w=w, wo=wo, co=co),
        grid=(n, ho // th),
        in_specs=[
            pl.BlockSpec((1, ho + 3, wo + 3, co), lambda b, r: (b, 0, 0, 0)),
            pl.BlockSpec((1, co), lambda b, r: (0, 0)),
            pl.BlockSpec((1, co), lambda b, r: (0, 0)),
            pl.BlockSpec((16 * co, co), lambda b, r: (0, 0)),
            pl.BlockSpec((1, h * w, ci), lambda b, r: (b, 0, 0)),
            pl.BlockSpec((ci, co), lambda b, r: (0, 0)),
            pl.BlockSpec((1, co), lambda b, r: (0, 0)),
        ],
        out_specs=pl.BlockSpec((1, co, th, wo), lambda b, r: (b, 0, r, 0)),
        out_shape=jax.ShapeDtypeStruct((n, co, ho, wo), jnp.float32),
        compiler_params=pltpu.CompilerParams(
            dimension_semantics=("parallel", "parallel")),
    )(y1p, scale2, shift2, w2m, xb16, wsm, bias)
    return out


# speedup vs baseline: 1.2595x; 1.2595x over previous
import jax
import jax.numpy as jnp
from jax.experimental import pallas as pl
from jax.experimental.pallas import tpu as pltpu

EPS_BN = 1e-5
SLOPE = 0.1

_OFFS = ((0, 1, 2), (1, 2))
_GROUPS = (((0,), (1, 2), (3,)), ((0, 1), (2, 3)))


def _leaky(v):
    return jnp.where(v >= 0, v, SLOPE * v)


def _stats_kernel(x_ref, s_ref, q_ref):
    xb = x_ref[0]
    s_ref[0, 0, :] = jnp.sum(xb, axis=1)
    q_ref[0, 0, :] = jnp.sum(xb * xb, axis=1)


def _prep_kernel(s1_ref, q1_ref, g1_ref, bt1_ref, u1_ref, wt1_ref,
                 u2_ref, w2t_ref, ws_ref, b2_ref, bs_ref,
                 sc1_ref, sh1_ref, m00_ref, m01_ref, m10_ref, m11_ref,
                 w2o_ref, wso_ref, b2o_ref, *, n, ci, hw):
    eps = 1e-12
    cnt = n * hw
    ssum = jnp.sum(s1_ref[...], axis=(0, 1))
    ssq = jnp.sum(q1_ref[...], axis=(0, 1))
    mean = ssum / cnt
    var = jnp.maximum(ssq / cnt - mean * mean, 0.0)
    inv = g1_ref[0] * jax.lax.rsqrt(var + EPS_BN)
    sc1_ref[0] = inv
    sh1_ref[0] = bt1_ref[0] - mean * inv

    def _inv_sigma(wt, u0):
        u = u0 / jnp.maximum(jnp.sqrt(jnp.sum(u0 * u0)), eps)
        v = jnp.sum(wt * u.reshape(1, -1), axis=1)
        v = v / jnp.maximum(jnp.sqrt(jnp.sum(v * v)), eps)
        wv = jnp.sum(wt * v.reshape(-1, 1), axis=0)
        u2 = wv / jnp.maximum(jnp.sqrt(jnp.sum(wv * wv)), eps)
        return 1.0 / jnp.sum(u2 * wv)

    wt1 = wt1_ref[...]
    wn1 = wt1 * _inv_sigma(wt1, u1_ref[0])
    mrefs = ((m00_ref, m01_ref), (m10_ref, m11_ref))
    for pa in (0, 1):
        for pb in (0, 1):
            blocks = []
            for gr in _GROUPS[pa]:
                for gc in _GROUPS[pb]:
                    blocks.append(
                        sum(wn1[(dy * 4 + dx) * ci:(dy * 4 + dx + 1) * ci]
                            for dy in gr for dx in gc))
            mrefs[pa][pb][...] = jnp.concatenate(
                blocks, axis=0).astype(jnp.bfloat16)
    w2t = w2t_ref[...]
    w2o_ref[...] = (w2t * _inv_sigma(w2t, u2_ref[0])).astype(jnp.bfloat16)
    wso_ref[...] = ws_ref[...].astype(jnp.bfloat16)
    b2o_ref[0] = b2_ref[0] + bs_ref[0]


def _conv1_kernel(x_ref, sc_ref, sh_ref, w00_ref, w01_ref, w10_ref, w11_ref,
                  b_ref, y_ref, xb_ref, s_ref, q_ref, *, h, w, co):
    xt = jnp.transpose(x_ref[0])
    xb_ref[0] = xt.astype(jnp.bfloat16)
    a = _leaky(xt * sc_ref[0].reshape(1, -1) + sh_ref[0].reshape(1, -1))
    a = a.astype(jnp.bfloat16).reshape(h, w, -1)
    a = jnp.concatenate([a[0:1], a, a[h - 1:h]], axis=0)
    a = jnp.concatenate([a[:, 0:1], a, a[:, w - 1:w]], axis=1)
    wrefs = ((w00_ref, w01_ref), (w10_ref, w11_ref))
    bias = b_ref[0].reshape(1, co)
    ssum = jnp.zeros((1, co), jnp.float32)
    ssq = jnp.zeros((1, co), jnp.float32)
    planes = [[None, None], [None, None]]
    for pa in (0, 1):
        for pb in (0, 1):
            pieces = [a[i:i + h, j:j + w, :]
                      for i in _OFFS[pa] for j in _OFFS[pb]]
            patch = jnp.concatenate(pieces, axis=-1).reshape(h * w, -1)
            acc = jnp.dot(patch, wrefs[pa][pb][...],
                          preferred_element_type=jnp.float32) + bias
            ssum = ssum + jnp.sum(acc, axis=0, keepdims=True)
            ssq = ssq + jnp.sum(acc * acc, axis=0, keepdims=True)
            planes[pa][pb] = acc.astype(jnp.bfloat16)
    s_ref[0] = ssum
    q_ref[0] = ssq
    c0 = jnp.concatenate([planes[0][0], planes[0][1]],
                         axis=1).reshape(h, 2 * w, co)
    c1 = jnp.concatenate([planes[1][0], planes[1][1]],
                         axis=1).reshape(h, 2 * w, co)
    z = jnp.concatenate([c0, c1], axis=1).reshape(2 * h, 2 * w, co)
    z = jnp.concatenate([z[:, 0:1], z, z[:, -1:], z[:, -1:]], axis=1)
    z = jnp.concatenate([z[0:1], z, z[-1:], z[-1:]], axis=0)
    y_ref[0] = z


def _conv2_kernel(y_ref, s2_ref, q2_ref, g2_ref, bt2_ref, w2_ref, xb_ref,
                  ws_ref, b_ref, o_ref, *, th, w, wo, co, cnt2):
    s2 = jnp.sum(s2_ref[...], axis=(0, 1))
    q2 = jnp.sum(q2_ref[...], axis=(0, 1))
    mean = s2 / cnt2
    var = jnp.maximum(q2 / cnt2 - mean * mean, 0.0)
    inv = g2_ref[0] * jax.lax.rsqrt(var + EPS_BN)
    sc = inv.reshape(1, 1, -1)
    sh = (bt2_ref[0] - mean * inv).reshape(1, 1, -1)

    r = pl.program_id(1)
    row0 = pl.multiple_of(r * th, th)
    yt = y_ref[0, pl.ds(row0, th + 3)]
    a = _leaky(yt.astype(jnp.float32) * sc + sh).astype(jnp.bfloat16)
    pieces = [a[i:i + th, j:j + wo, :] for i in range(4) for j in range(4)]
    patch = jnp.concatenate(pieces, axis=-1).reshape(th * wo, -1)
    acc = jnp.dot(patch, w2_ref[...], preferred_element_type=jnp.float32)
    hs = th // 2
    xs = xb_ref[0, pl.ds(r * hs * w, hs * w)].reshape(hs, w, -1)
    xs = jnp.stack([xs, xs], axis=1).reshape(th, w, -1)
    xs = jnp.stack([xs, xs], axis=2).reshape(th, wo, -1).reshape(th * wo, -1)
    acc = acc + jnp.dot(xs, ws_ref[...], preferred_element_type=jnp.float32)
    acc = acc + b_ref[0].reshape(1, co)
    o_ref[0] = acc.T.astype(jnp.float32)


def kernel(x, bn1_gamma, bn1_beta, w1, b1, u1,
           bn2_gamma, bn2_beta, w2, b2, u2, ws, bs):
    n, ci, h, w = x.shape
    co = w1.shape[0]
    ho, wo = 2 * h, 2 * w
    xf = x.astype(jnp.float32).reshape(n, ci, h * w)

    s1p, q1p = pl.pallas_call(
        _stats_kernel,
        grid=(n,),
        in_specs=[pl.BlockSpec((1, ci, h * w), lambda i: (i, 0, 0))],
        out_specs=(pl.BlockSpec((1, 1, ci), lambda i: (i, 0, 0)),
                   pl.BlockSpec((1, 1, ci), lambda i: (i, 0, 0))),
        out_shape=(jax.ShapeDtypeStruct((n, 1, ci), jnp.float32),
                   jax.ShapeDtypeStruct((n, 1, ci), jnp.float32)),
        compiler_params=pltpu.CompilerParams(
            dimension_semantics=("parallel",)),
    )(xf)

    wt1r = jnp.transpose(w1, (2, 3, 1, 0)).reshape(16 * ci, co)
    w2tr = jnp.transpose(w2, (2, 3, 1, 0)).reshape(16 * co, co)
    wsr = ws[:, :, 0, 0].T
    vec = lambda a, m: pl.BlockSpec((1, m), lambda i: (0, 0))
    prep = pl.pallas_call(
        lambda *refs: _prep_kernel(*refs, n=n, ci=ci, hw=h * w),
        grid=(1,),
        in_specs=[
            pl.BlockSpec((n, 1, ci), lambda i: (0, 0, 0)),
            pl.BlockSpec((n, 1, ci), lambda i: (0, 0, 0)),
            vec(None, ci), vec(None, ci), vec(None, co),
            pl.BlockSpec((16 * ci, co), lambda i: (0, 0)),
            vec(None, co),
            pl.BlockSpec((16 * co, co), lambda i: (0, 0)),
            pl.BlockSpec((ci, co), lambda i: (0, 0)),
            vec(None, co), vec(None, co),
        ],
        out_specs=(
            vec(None, ci), vec(None, ci),
            pl.BlockSpec((9 * ci, co), lambda i: (0, 0)),
            pl.BlockSpec((6 * ci, co), lambda i: (0, 0)),
            pl.BlockSpec((6 * ci, co), lambda i: (0, 0)),
            pl.BlockSpec((4 * ci, co), lambda i: (0, 0)),
            pl.BlockSpec((16 * co, co), lambda i: (0, 0)),
            pl.BlockSpec((ci, co), lambda i: (0, 0)),
            vec(None, co),
        ),
        out_shape=(
            jax.ShapeDtypeStruct((1, ci), jnp.float32),
            jax.ShapeDtypeStruct((1, ci), jnp.float32),
            jax.ShapeDtypeStruct((9 * ci, co), jnp.bfloat16),
            jax.ShapeDtypeStruct((6 * ci, co), jnp.bfloat16),
            jax.ShapeDtypeStruct((6 * ci, co), jnp.bfloat16),
            jax.ShapeDtypeStruct((4 * ci, co), jnp.bfloat16),
            jax.ShapeDtypeStruct((16 * co, co), jnp.bfloat16),
            jax.ShapeDtypeStruct((ci, co), jnp.bfloat16),
            jax.ShapeDtypeStruct((1, co), jnp.float32),
        ),
        compiler_params=pltpu.CompilerParams(
            dimension_semantics=("arbitrary",)),
    )
    (scale1, shift1, m00, m01, m10, m11, w2m, wsm, bias) = prep(
        s1p, q1p, bn1_gamma.reshape(1, ci), bn1_beta.reshape(1, ci),
        u1.reshape(1, co), wt1r, u2.reshape(1, co), w2tr, wsr,
        b2.reshape(1, co), bs.reshape(1, co))

    conv1 = pl.pallas_call(
        lambda *refs: _conv1_kernel(*refs, h=h, w=w, co=co),
        grid=(n,),
        in_specs=[
            pl.BlockSpec((1, ci, h * w), lambda b: (b, 0, 0)),
            pl.BlockSpec((1, ci), lambda b: (0, 0)),
            pl.BlockSpec((1, ci), lambda b: (0, 0)),
            pl.BlockSpec((9 * ci, co), lambda b: (0, 0)),
            pl.BlockSpec((6 * ci, co), lambda b: (0, 0)),
            pl.BlockSpec((6 * ci, co), lambda b: (0, 0)),
            pl.BlockSpec((4 * ci, co), lambda b: (0, 0)),
            pl.BlockSpec((1, co), lambda b: (0, 0)),
        ],
        out_specs=(
            pl.BlockSpec((1, ho + 3, wo + 3, co), lambda b: (b, 0, 0, 0)),
            pl.BlockSpec((1, h * w, ci), lambda b: (b, 0, 0)),
            pl.BlockSpec((1, 1, co), lambda b: (b, 0, 0)),
            pl.BlockSpec((1, 1, co), lambda b: (b, 0, 0)),
        ),
        out_shape=(
            jax.ShapeDtypeStruct((n, ho + 3, wo + 3, co), jnp.bfloat16),
            jax.ShapeDtypeStruct((n, h * w, ci), jnp.bfloat16),
            jax.ShapeDtypeStruct((n, 1, co), jnp.float32),
            jax.ShapeDtypeStruct((n, 1, co), jnp.float32),
        ),
        compiler_params=pltpu.CompilerParams(
            dimension_semantics=("parallel",)),
    )
    y1p, xb16, s2p, q2p = conv1(xf, scale1, shift1, m00, m01, m10, m11,
                                b1.reshape(1, co))

    th = 16 if ho % 16 == 0 else ho
    out = pl.pallas_call(
        lambda *refs: _conv2_kernel(*refs, th=th, w=w, wo=wo, co=co,
                                    cnt2=n * ho * wo),
        grid=(n, ho // th),
        in_specs=[
            pl.BlockSpec((1, ho + 3, wo + 3, co), lambda b, r: (b, 0, 0, 0)),
            pl.BlockSpec((n, 1, co), lambda b, r: (0, 0, 0)),
            pl.BlockSpec((n, 1, co), lambda b, r: (0, 0, 0)),
            pl.BlockSpec((1, co), lambda b, r: (0, 0)),
            pl.BlockSpec((1, co), lambda b, r: (0, 0)),
            pl.BlockSpec((16 * co, co), lambda b, r: (0, 0)),
            pl.BlockSpec((1, h * w, ci), lambda b, r: (b, 0, 0)),
            pl.BlockSpec((ci, co), lambda b, r: (0, 0)),
            pl.BlockSpec((1, co), lambda b, r: (0, 0)),
        ],
        out_specs=pl.BlockSpec((1, co, th * wo), lambda b, r: (b, 0, r)),
        out_shape=jax.ShapeDtypeStruct((n, co, ho * wo), jnp.float32),
        compiler_params=pltpu.CompilerParams(
            dimension_semantics=("parallel", "parallel")),
    )(y1p, s2p, q2p, bn2_gamma.reshape(1, co), bn2_beta.reshape(1, co),
      w2m, xb16, wsm, bias)
    return out.reshape(n, co, ho, wo)


# speedup vs baseline: 1.3229x; 1.0503x over previous
import jax
import jax.numpy as jnp
from jax.experimental import pallas as pl
from jax.experimental.pallas import tpu as pltpu

EPS_BN = 1e-5
SLOPE = 0.1

_OFFS = ((0, 1, 2), (1, 2))
_GROUPS = (((0,), (1, 2), (3,)), ((0, 1), (2, 3)))


def _leaky(v):
    return jnp.where(v >= 0, v, SLOPE * v)


def _stats_kernel(x_ref, s_ref, q_ref):
    xb = x_ref[0]
    s_ref[0, 0, :] = jnp.sum(xb, axis=1)
    q_ref[0, 0, :] = jnp.sum(xb * xb, axis=1)


def _prep_kernel(s1_ref, q1_ref, g1_ref, bt1_ref, u1_ref, wt1_ref,
                 u2_ref, w2t_ref, ws_ref, b2_ref, bs_ref,
                 sc1_ref, sh1_ref, m00_ref, m01_ref, m10_ref, m11_ref,
                 w2o_ref, wso_ref, b2o_ref, *, n, ci, hw):
    eps = 1e-12
    cnt = n * hw
    ssum = jnp.sum(s1_ref[...], axis=(0, 1))
    ssq = jnp.sum(q1_ref[...], axis=(0, 1))
    mean = ssum / cnt
    var = jnp.maximum(ssq / cnt - mean * mean, 0.0)
    inv = g1_ref[0] * jax.lax.rsqrt(var + EPS_BN)
    sc1_ref[0] = inv
    sh1_ref[0] = bt1_ref[0] - mean * inv

    def _inv_sigma(wt, u0):
        u = u0 / jnp.maximum(jnp.sqrt(jnp.sum(u0 * u0)), eps)
        v = jnp.sum(wt * u.reshape(1, -1), axis=1)
        v = v / jnp.maximum(jnp.sqrt(jnp.sum(v * v)), eps)
        wv = jnp.sum(wt * v.reshape(-1, 1), axis=0)
        u2 = wv / jnp.maximum(jnp.sqrt(jnp.sum(wv * wv)), eps)
        return 1.0 / jnp.sum(u2 * wv)

    wt1 = wt1_ref[...]
    wn1 = wt1 * _inv_sigma(wt1, u1_ref[0])
    mrefs = ((m00_ref, m01_ref), (m10_ref, m11_ref))
    for pa in (0, 1):
        for pb in (0, 1):
            blocks = []
            for gr in _GROUPS[pa]:
                for gc in _GROUPS[pb]:
                    blocks.append(
                        sum(wn1[(dy * 4 + dx) * ci:(dy * 4 + dx + 1) * ci]
                            for dy in gr for dx in gc))
            mrefs[pa][pb][...] = jnp.concatenate(
                blocks, axis=0).astype(jnp.bfloat16)
    w2t = w2t_ref[...]
    w2o_ref[...] = (w2t * _inv_sigma(w2t, u2_ref[0])).astype(jnp.bfloat16)
    wso_ref[...] = ws_ref[...].astype(jnp.bfloat16)
    b2o_ref[0] = b2_ref[0] + bs_ref[0]


def _conv1_kernel(x_ref, sc_ref, sh_ref, w00_ref, w01_ref, w10_ref, w11_ref,
                  b_ref, y_ref, xb_ref, s_ref, q_ref, *, h, w, co):
    xt = jnp.transpose(x_ref[0])
    xb_ref[0] = xt.astype(jnp.bfloat16)
    a = _leaky(xt * sc_ref[0].reshape(1, -1) + sh_ref[0].reshape(1, -1))
    a = a.astype(jnp.bfloat16).reshape(h, w, -1)
    a = jnp.concatenate([a[0:1], a, a[h - 1:h]], axis=0)
    a = jnp.concatenate([a[:, 0:1], a, a[:, w - 1:w]], axis=1)
    wrefs = ((w00_ref, w01_ref), (w10_ref, w11_ref))
    bias = b_ref[0].reshape(1, co)
    ssum = jnp.zeros((1, co), jnp.float32)
    ssq = jnp.zeros((1, co), jnp.float32)
    planes = [[None, None], [None, None]]
    for pa in (0, 1):
        for pb in (0, 1):
            pieces = [a[i:i + h, j:j + w, :]
                      for i in _OFFS[pa] for j in _OFFS[pb]]
            patch = jnp.concatenate(pieces, axis=-1).reshape(h * w, -1)
            acc = jnp.dot(patch, wrefs[pa][pb][...],
                          preferred_element_type=jnp.float32) + bias
            ssum = ssum + jnp.sum(acc, axis=0, keepdims=True)
            ssq = ssq + jnp.sum(acc * acc, axis=0, keepdims=True)
            planes[pa][pb] = acc.astype(jnp.bfloat16)
    s_ref[0] = ssum
    q_ref[0] = ssq
    c0 = jnp.concatenate([planes[0][0], planes[0][1]],
                         axis=1).reshape(h, 2 * w, co)
    c1 = jnp.concatenate([planes[1][0], planes[1][1]],
                         axis=1).reshape(h, 2 * w, co)
    z = jnp.concatenate([c0, c1], axis=1).reshape(2 * h, 2 * w, co)
    z = jnp.concatenate([z[:, 0:1], z, z[:, -1:], z[:, -1:]], axis=1)
    z = jnp.concatenate([z[0:1], z, z[-1:], z[-1:]], axis=0)
    y_ref[0] = z


def _conv2_kernel(y_ref, s2_ref, q2_ref, g2_ref, bt2_ref, w2_ref, xb_ref,
                  ws_ref, b_ref, o_ref, *, th, w, wo, co, cnt2):
    s2 = jnp.sum(s2_ref[...], axis=(0, 1))
    q2 = jnp.sum(q2_ref[...], axis=(0, 1))
    mean = s2 / cnt2
    var = jnp.maximum(q2 / cnt2 - mean * mean, 0.0)
    inv = g2_ref[0] * jax.lax.rsqrt(var + EPS_BN)
    sc = inv.reshape(1, 1, -1)
    sh = (bt2_ref[0] - mean * inv).reshape(1, 1, -1)

    r = pl.program_id(1)
    row0 = pl.multiple_of(r * th, th)
    yt = y_ref[0, pl.ds(row0, th + 3)]
    a = _leaky(yt.astype(jnp.float32) * sc + sh).astype(jnp.bfloat16)
    pieces = [a[i:i + th, j:j + wo, :] for i in range(4) for j in range(4)]
    patch = jnp.concatenate(pieces, axis=-1).reshape(th * wo, -1)
    acc = jnp.dot(patch, w2_ref[...], preferred_element_type=jnp.float32)
    hs = th // 2
    xs = xb_ref[0, pl.ds(r * hs * w, hs * w)].reshape(hs, w, -1)
    xs = jnp.concatenate([xs, xs], axis=1).reshape(th, w, -1)
    xs = jnp.concatenate([xs, xs], axis=2).reshape(th, wo, -1)
    xs = xs.reshape(th * wo, -1)
    acc = acc + jnp.dot(xs, ws_ref[...], preferred_element_type=jnp.float32)
    acc = acc + b_ref[0].reshape(1, co)
    o_ref[0] = acc.T.astype(jnp.float32)


def kernel(x, bn1_gamma, bn1_beta, w1, b1, u1,
           bn2_gamma, bn2_beta, w2, b2, u2, ws, bs):
    n, ci, h, w = x.shape
    co = w1.shape[0]
    ho, wo = 2 * h, 2 * w
    xf = x.astype(jnp.float32).reshape(n, ci, h * w)

    s1p, q1p = pl.pallas_call(
        _stats_kernel,
        grid=(n,),
        in_specs=[pl.BlockSpec((1, ci, h * w), lambda i: (i, 0, 0))],
        out_specs=(pl.BlockSpec((1, 1, ci), lambda i: (i, 0, 0)),
                   pl.BlockSpec((1, 1, ci), lambda i: (i, 0, 0))),
        out_shape=(jax.ShapeDtypeStruct((n, 1, ci), jnp.float32),
                   jax.ShapeDtypeStruct((n, 1, ci), jnp.float32)),
        compiler_params=pltpu.CompilerParams(
            dimension_semantics=("parallel",)),
    )(xf)

    wt1r = jnp.transpose(w1, (2, 3, 1, 0)).reshape(16 * ci, co)
    w2tr = jnp.transpose(w2, (2, 3, 1, 0)).reshape(16 * co, co)
    wsr = ws[:, :, 0, 0].T
    vec = lambda a, m: pl.BlockSpec((1, m), lambda i: (0, 0))
    prep = pl.pallas_call(
        lambda *refs: _prep_kernel(*refs, n=n, ci=ci, hw=h * w),
        grid=(1,),
        in_specs=[
            pl.BlockSpec((n, 1, ci), lambda i: (0, 0, 0)),
            pl.BlockSpec((n, 1, ci), lambda i: (0, 0, 0)),
            vec(None, ci), vec(None, ci), vec(None, co),
            pl.BlockSpec((16 * ci, co), lambda i: (0, 0)),
            vec(None, co),
            pl.BlockSpec((16 * co, co), lambda i: (0, 0)),
            pl.BlockSpec((ci, co), lambda i: (0, 0)),
            vec(None, co), vec(None, co),
        ],
        out_specs=(
            vec(None, ci), vec(None, ci),
            pl.BlockSpec((9 * ci, co), lambda i: (0, 0)),
            pl.BlockSpec((6 * ci, co), lambda i: (0, 0)),
            pl.BlockSpec((6 * ci, co), lambda i: (0, 0)),
            pl.BlockSpec((4 * ci, co), lambda i: (0, 0)),
            pl.BlockSpec((16 * co, co), lambda i: (0, 0)),
            pl.BlockSpec((ci, co), lambda i: (0, 0)),
            vec(None, co),
        ),
        out_shape=(
            jax.ShapeDtypeStruct((1, ci), jnp.float32),
            jax.ShapeDtypeStruct((1, ci), jnp.float32),
            jax.ShapeDtypeStruct((9 * ci, co), jnp.bfloat16),
            jax.ShapeDtypeStruct((6 * ci, co), jnp.bfloat16),
            jax.ShapeDtypeStruct((6 * ci, co), jnp.bfloat16),
            jax.ShapeDtypeStruct((4 * ci, co), jnp.bfloat16),
            jax.ShapeDtypeStruct((16 * co, co), jnp.bfloat16),
            jax.ShapeDtypeStruct((ci, co), jnp.bfloat16),
            jax.ShapeDtypeStruct((1, co), jnp.float32),
        ),
        compiler_params=pltpu.CompilerParams(
            dimension_semantics=("arbitrary",)),
    )
    (scale1, shift1, m00, m01, m10, m11, w2m, wsm, bias) = prep(
        s1p, q1p, bn1_gamma.reshape(1, ci), bn1_beta.reshape(1, ci),
        u1.reshape(1, co), wt1r, u2.reshape(1, co), w2tr, wsr,
        b2.reshape(1, co), bs.reshape(1, co))

    conv1 = pl.pallas_call(
        lambda *refs: _conv1_kernel(*refs, h=h, w=w, co=co),
        grid=(n,),
        in_specs=[
            pl.BlockSpec((1, ci, h * w), lambda b: (b, 0, 0)),
            pl.BlockSpec((1, ci), lambda b: (0, 0)),
            pl.BlockSpec((1, ci), lambda b: (0, 0)),
            pl.BlockSpec((9 * ci, co), lambda b: (0, 0)),
            pl.BlockSpec((6 * ci, co), lambda b: (0, 0)),
            pl.BlockSpec((6 * ci, co), lambda b: (0, 0)),
            pl.BlockSpec((4 * ci, co), lambda b: (0, 0)),
            pl.BlockSpec((1, co), lambda b: (0, 0)),
        ],
        out_specs=(
            pl.BlockSpec((1, ho + 3, wo + 3, co), lambda b: (b, 0, 0, 0)),
            pl.BlockSpec((1, h * w, ci), lambda b: (b, 0, 0)),
            pl.BlockSpec((1, 1, co), lambda b: (b, 0, 0)),
            pl.BlockSpec((1, 1, co), lambda b: (b, 0, 0)),
        ),
        out_shape=(
            jax.ShapeDtypeStruct((n, ho + 3, wo + 3, co), jnp.bfloat16),
            jax.ShapeDtypeStruct((n, h * w, ci), jnp.bfloat16),
            jax.ShapeDtypeStruct((n, 1, co), jnp.float32),
            jax.ShapeDtypeStruct((n, 1, co), jnp.float32),
        ),
        compiler_params=pltpu.CompilerParams(
            dimension_semantics=("parallel",)),
    )
    y1p, xb16, s2p, q2p = conv1(xf, scale1, shift1, m00, m01, m10, m11,
                                b1.reshape(1, co))

    th = 32 if ho % 32 == 0 else ho
    out = pl.pallas_call(
        lambda *refs: _conv2_kernel(*refs, th=th, w=w, wo=wo, co=co,
                                    cnt2=n * ho * wo),
        grid=(n, ho // th),
        in_specs=[
            pl.BlockSpec((1, ho + 3, wo + 3, co), lambda b, r: (b, 0, 0, 0)),
            pl.BlockSpec((n, 1, co), lambda b, r: (0, 0, 0)),
            pl.BlockSpec((n, 1, co), lambda b, r: (0, 0, 0)),
            pl.BlockSpec((1, co), lambda b, r: (0, 0)),
            pl.BlockSpec((1, co), lambda b, r: (0, 0)),
            pl.BlockSpec((16 * co, co), lambda b, r: (0, 0)),
            pl.BlockSpec((1, h * w, ci), lambda b, r: (b, 0, 0)),
            pl.BlockSpec((ci, co), lambda b, r: (0, 0)),
            pl.BlockSpec((1, co), lambda b, r: (0, 0)),
        ],
        out_specs=pl.BlockSpec((1, co, th * wo), lambda b, r: (b, 0, r)),
        out_shape=jax.ShapeDtypeStruct((n, co, ho * wo), jnp.float32),
        compiler_params=pltpu.CompilerParams(
            dimension_semantics=("parallel", "parallel")),
    )(y1p, s2p, q2p, bn2_gamma.reshape(1, co), bn2_beta.reshape(1, co),
      w2m, xb16, wsm, bias)
    return out.reshape(n, co, ho, wo)


# speedup vs baseline: 1.3328x; 1.0075x over previous
import jax
import jax.numpy as jnp
from jax.experimental import pallas as pl
from jax.experimental.pallas import tpu as pltpu

EPS_BN = 1e-5
SLOPE = 0.1

_OFFS = ((0, 1, 2), (1, 2))
_GROUPS = (((0,), (1, 2), (3,)), ((0, 1), (2, 3)))


def _leaky(v):
    return jnp.where(v >= 0, v, SLOPE * v)


def _stats_kernel(x_ref, s_ref, q_ref):
    xb = x_ref[0]
    s_ref[0, 0, :] = jnp.sum(xb, axis=1)
    q_ref[0, 0, :] = jnp.sum(xb * xb, axis=1)


def _prep_kernel(s1_ref, q1_ref, g1_ref, bt1_ref, u1_ref, wt1_ref,
                 u2_ref, w2t_ref, ws_ref, b2_ref, bs_ref,
                 sc1_ref, sh1_ref, m00_ref, m01_ref, m10_ref, m11_ref,
                 w2o_ref, wso_ref, b2o_ref, *, n, ci, hw):
    eps = 1e-12
    cnt = n * hw
    ssum = jnp.sum(s1_ref[...], axis=(0, 1))
    ssq = jnp.sum(q1_ref[...], axis=(0, 1))
    mean = ssum / cnt
    var = jnp.maximum(ssq / cnt - mean * mean, 0.0)
    inv = g1_ref[0] * jax.lax.rsqrt(var + EPS_BN)
    sc1_ref[0] = inv
    sh1_ref[0] = bt1_ref[0] - mean * inv

    def _inv_sigma(wt, u0):
        u = u0 / jnp.maximum(jnp.sqrt(jnp.sum(u0 * u0)), eps)
        v = jnp.sum(wt * u.reshape(1, -1), axis=1)
        v = v / jnp.maximum(jnp.sqrt(jnp.sum(v * v)), eps)
        wv = jnp.sum(wt * v.reshape(-1, 1), axis=0)
        u2 = wv / jnp.maximum(jnp.sqrt(jnp.sum(wv * wv)), eps)
        return 1.0 / jnp.sum(u2 * wv)

    wt1 = wt1_ref[...]
    wn1 = wt1 * _inv_sigma(wt1, u1_ref[0])
    mrefs = ((m00_ref, m01_ref), (m10_ref, m11_ref))
    for pa in (0, 1):
        for pb in (0, 1):
            blocks = []
            for gr in _GROUPS[pa]:
                for gc in _GROUPS[pb]:
                    blocks.append(
                        sum(wn1[(dy * 4 + dx) * ci:(dy * 4 + dx + 1) * ci]
                            for dy in gr for dx in gc))
            mrefs[pa][pb][...] = jnp.concatenate(
                blocks, axis=0).astype(jnp.bfloat16)
    w2t = w2t_ref[...]
    w2o_ref[...] = (w2t * _inv_sigma(w2t, u2_ref[0])).astype(jnp.bfloat16)
    wso_ref[...] = ws_ref[...].astype(jnp.bfloat16)
    b2o_ref[0] = b2_ref[0] + bs_ref[0]


def _conv1_kernel(x_ref, sc_ref, sh_ref, w00_ref, w01_ref, w10_ref, w11_ref,
                  b_ref, y_ref, xb_ref, s_ref, q_ref, *, h, w, co):
    xt = jnp.transpose(x_ref[0])
    xb_ref[0] = xt.astype(jnp.bfloat16)
    a = _leaky(xt * sc_ref[0].reshape(1, -1) + sh_ref[0].reshape(1, -1))
    a = a.astype(jnp.bfloat16).reshape(h, w, -1)
    a = jnp.concatenate([a[0:1], a, a[h - 1:h]], axis=0)
    a = jnp.concatenate([a[:, 0:1], a, a[:, w - 1:w]], axis=1)
    wrefs = ((w00_ref, w01_ref), (w10_ref, w11_ref))
    bias = b_ref[0].reshape(1, co)
    ssum = jnp.zeros((1, co), jnp.float32)
    ssq = jnp.zeros((1, co), jnp.float32)
    planes = [[None, None], [None, None]]
    for pa in (0, 1):
        for pb in (0, 1):
            pieces = [a[i:i + h, j:j + w, :]
                      for i in _OFFS[pa] for j in _OFFS[pb]]
            patch = jnp.concatenate(pieces, axis=-1).reshape(h * w, -1)
            acc = jnp.dot(patch, wrefs[pa][pb][...],
                          preferred_element_type=jnp.float32) + bias
            ssum = ssum + jnp.sum(acc, axis=0, keepdims=True)
            ssq = ssq + jnp.sum(acc * acc, axis=0, keepdims=True)
            planes[pa][pb] = acc.astype(jnp.bfloat16)
    s_ref[0] = ssum
    q_ref[0] = ssq
    c0 = jnp.concatenate([planes[0][0], planes[0][1]],
                         axis=1).reshape(h, 2 * w, co)
    c1 = jnp.concatenate([planes[1][0], planes[1][1]],
                         axis=1).reshape(h, 2 * w, co)
    z = jnp.concatenate([c0, c1], axis=1).reshape(2 * h, 2 * w, co)
    z = jnp.concatenate([z[:, 0:1], z, z[:, -1:], z[:, -1:]], axis=1)
    z = jnp.concatenate([z[0:1], z, z[-1:], z[-1:]], axis=0)
    y_ref[0] = z


def _conv2_kernel(y_ref, s2_ref, q2_ref, g2_ref, bt2_ref, w2_ref, xb_ref,
                  ws_ref, b_ref, o_ref, *, th, w, wo, co, cnt2):
    s2 = jnp.sum(s2_ref[...], axis=(0, 1))
    q2 = jnp.sum(q2_ref[...], axis=(0, 1))
    mean = s2 / cnt2
    var = jnp.maximum(q2 / cnt2 - mean * mean, 0.0)
    inv = g2_ref[0] * jax.lax.rsqrt(var + EPS_BN)
    sc = inv.reshape(1, 1, -1).astype(jnp.bfloat16)
    sh = (bt2_ref[0] - mean * inv).reshape(1, 1, -1).astype(jnp.bfloat16)

    r = pl.program_id(1)
    row0 = pl.multiple_of(r * th, th)
    yt = y_ref[0, pl.ds(row0, th + 3)]
    a = _leaky(yt * sc + sh)
    pieces = [a[i:i + th, j:j + wo, :] for i in range(4) for j in range(4)]
    patch = jnp.concatenate(pieces, axis=-1).reshape(th * wo, -1)
    acc = jnp.dot(patch, w2_ref[...], preferred_element_type=jnp.float32)
    hs = th // 2
    xs = xb_ref[0, pl.ds(r * hs * w, hs * w)].reshape(hs, w, -1)
    xs = jnp.concatenate([xs, xs], axis=1).reshape(th, w, -1)
    xs = jnp.concatenate([xs, xs], axis=2).reshape(th, wo, -1)
    xs = xs.reshape(th * wo, -1)
    acc = acc + jnp.dot(xs, ws_ref[...], preferred_element_type=jnp.float32)
    acc = acc + b_ref[0].reshape(1, co)
    o_ref[0] = acc.T.astype(jnp.float32)


def kernel(x, bn1_gamma, bn1_beta, w1, b1, u1,
           bn2_gamma, bn2_beta, w2, b2, u2, ws, bs):
    n, ci, h, w = x.shape
    co = w1.shape[0]
    ho, wo = 2 * h, 2 * w
    xf = x.astype(jnp.float32).reshape(n, ci, h * w)

    s1p, q1p = pl.pallas_call(
        _stats_kernel,
        grid=(n,),
        in_specs=[pl.BlockSpec((1, ci, h * w), lambda i: (i, 0, 0))],
        out_specs=(pl.BlockSpec((1, 1, ci), lambda i: (i, 0, 0)),
                   pl.BlockSpec((1, 1, ci), lambda i: (i, 0, 0))),
        out_shape=(jax.ShapeDtypeStruct((n, 1, ci), jnp.float32),
                   jax.ShapeDtypeStruct((n, 1, ci), jnp.float32)),
        compiler_params=pltpu.CompilerParams(
            dimension_semantics=("parallel",)),
    )(xf)

    wt1r = jnp.transpose(w1, (2, 3, 1, 0)).reshape(16 * ci, co)
    w2tr = jnp.transpose(w2, (2, 3, 1, 0)).reshape(16 * co, co)
    wsr = ws[:, :, 0, 0].T
    vec = lambda a, m: pl.BlockSpec((1, m), lambda i: (0, 0))
    prep = pl.pallas_call(
        lambda *refs: _prep_kernel(*refs, n=n, ci=ci, hw=h * w),
        grid=(1,),
        in_specs=[
            pl.BlockSpec((n, 1, ci), lambda i: (0, 0, 0)),
            pl.BlockSpec((n, 1, ci), lambda i: (0, 0, 0)),
            vec(None, ci), vec(None, ci), vec(None, co),
            pl.BlockSpec((16 * ci, co), lambda i: (0, 0)),
            vec(None, co),
            pl.BlockSpec((16 * co, co), lambda i: (0, 0)),
            pl.BlockSpec((ci, co), lambda i: (0, 0)),
            vec(None, co), vec(None, co),
        ],
        out_specs=(
            vec(None, ci), vec(None, ci),
            pl.BlockSpec((9 * ci, co), lambda i: (0, 0)),
            pl.BlockSpec((6 * ci, co), lambda i: (0, 0)),
            pl.BlockSpec((6 * ci, co), lambda i: (0, 0)),
            pl.BlockSpec((4 * ci, co), lambda i: (0, 0)),
            pl.BlockSpec((16 * co, co), lambda i: (0, 0)),
            pl.BlockSpec((ci, co), lambda i: (0, 0)),
            vec(None, co),
        ),
        out_shape=(
            jax.ShapeDtypeStruct((1, ci), jnp.float32),
            jax.ShapeDtypeStruct((1, ci), jnp.float32),
            jax.ShapeDtypeStruct((9 * ci, co), jnp.bfloat16),
            jax.ShapeDtypeStruct((6 * ci, co), jnp.bfloat16),
            jax.ShapeDtypeStruct((6 * ci, co), jnp.bfloat16),
            jax.ShapeDtypeStruct((4 * ci, co), jnp.bfloat16),
            jax.ShapeDtypeStruct((16 * co, co), jnp.bfloat16),
            jax.ShapeDtypeStruct((ci, co), jnp.bfloat16),
            jax.ShapeDtypeStruct((1, co), jnp.float32),
        ),
        compiler_params=pltpu.CompilerParams(
            dimension_semantics=("arbitrary",)),
    )
    (scale1, shift1, m00, m01, m10, m11, w2m, wsm, bias) = prep(
        s1p, q1p, bn1_gamma.reshape(1, ci), bn1_beta.reshape(1, ci),
        u1.reshape(1, co), wt1r, u2.reshape(1, co), w2tr, wsr,
        b2.reshape(1, co), bs.reshape(1, co))

    conv1 = pl.pallas_call(
        lambda *refs: _conv1_kernel(*refs, h=h, w=w, co=co),
        grid=(n,),
        in_specs=[
            pl.BlockSpec((1, ci, h * w), lambda b: (b, 0, 0)),
            pl.BlockSpec((1, ci), lambda b: (0, 0)),
            pl.BlockSpec((1, ci), lambda b: (0, 0)),
            pl.BlockSpec((9 * ci, co), lambda b: (0, 0)),
            pl.BlockSpec((6 * ci, co), lambda b: (0, 0)),
            pl.BlockSpec((6 * ci, co), lambda b: (0, 0)),
            pl.BlockSpec((4 * ci, co), lambda b: (0, 0)),
            pl.BlockSpec((1, co), lambda b: (0, 0)),
        ],
        out_specs=(
            pl.BlockSpec((1, ho + 3, wo + 3, co), lambda b: (b, 0, 0, 0)),
            pl.BlockSpec((1, h * w, ci), lambda b: (b, 0, 0)),
            pl.BlockSpec((1, 1, co), lambda b: (b, 0, 0)),
            pl.BlockSpec((1, 1, co), lambda b: (b, 0, 0)),
        ),
        out_shape=(
            jax.ShapeDtypeStruct((n, ho + 3, wo + 3, co), jnp.bfloat16),
            jax.ShapeDtypeStruct((n, h * w, ci), jnp.bfloat16),
            jax.ShapeDtypeStruct((n, 1, co), jnp.float32),
            jax.ShapeDtypeStruct((n, 1, co), jnp.float32),
        ),
        compiler_params=pltpu.CompilerParams(
            dimension_semantics=("parallel",)),
    )
    y1p, xb16, s2p, q2p = conv1(xf, scale1, shift1, m00, m01, m10, m11,
                                b1.reshape(1, co))

    th = 32 if ho % 32 == 0 else ho
    out = pl.pallas_call(
        lambda *refs: _conv2_kernel(*refs, th=th, w=w, wo=wo, co=co,
                                    cnt2=n * ho * wo),
        grid=(n, ho // th),
        in_specs=[
            pl.BlockSpec((1, ho + 3, wo + 3, co), lambda b, r: (b, 0, 0, 0)),
            pl.BlockSpec((n, 1, co), lambda b, r: (0, 0, 0)),
            pl.BlockSpec((n, 1, co), lambda b, r: (0, 0, 0)),
            pl.BlockSpec((1, co), lambda b, r: (0, 0)),
            pl.BlockSpec((1, co), lambda b, r: (0, 0)),
            pl.BlockSpec((16 * co, co), lambda b, r: (0, 0)),
            pl.BlockSpec((1, h * w, ci), lambda b, r: (b, 0, 0)),
            pl.BlockSpec((ci, co), lambda b, r: (0, 0)),
            pl.BlockSpec((1, co), lambda b, r: (0, 0)),
        ],
        out_specs=pl.BlockSpec((1, co, th * wo), lambda b, r: (b, 0, r)),
        out_shape=jax.ShapeDtypeStruct((n, co, ho * wo), jnp.float32),
        compiler_params=pltpu.CompilerParams(
            dimension_semantics=("parallel", "parallel")),
    )(y1p, s2p, q2p, bn2_gamma.reshape(1, co), bn2_beta.reshape(1, co),
      w2m, xb16, wsm, bias)
    return out.reshape(n, co, ho, wo)
